# baseline (speedup 1.0000x reference)
import functools

import jax
import jax.numpy as jnp
from jax import lax
from jax.experimental import pallas as pl
from jax.experimental.pallas import tpu as pltpu
from jax.experimental.pallas import tpu_sc as plsc

_N = 10000
_E = 320000
_DIN = 128
_D = 256
_NE = 4
_STEPS = 8
_NG = 20
_NPG = 500
_DC = _DIN + _D

_H = 128
_NCORE = 2
_NSUB = 16
_EPT = _E // _NSUB
_CH = 80
_NCHK = _EPT // _CH
_NPR = _NCHK // 2
_RPT = 624
_RREM = _N - _NSUB * _RPT
_RREM_OFF = _NSUB * _RPT

_RB = 2000
_NRB = _N // _RB


def _msg_body(h_ref, w_ref, b_ref, out_ref):
    w = w_ref[0]
    acc = lax.dot_general(h_ref[...], w, (((1,), (1,)), ((), ())),
                          preferred_element_type=jnp.float32)
    out_ref[...] = acc + b_ref[0]


def _msg_call(h, W_msg, b_msg3):
    return pl.pallas_call(
        _msg_body,
        grid=(_NRB, _NCORE, _NE),
        in_specs=[
            pl.BlockSpec((_RB, _D), lambda r, c, e: (r, 0)),
            pl.BlockSpec((1, _H, _D), lambda r, c, e: (e, c, 0)),
            pl.BlockSpec((1, 1, _H), lambda r, c, e: (e * _NCORE + c, 0, 0)),
        ],
        out_specs=pl.BlockSpec((_RB, _H), lambda r, c, e: (c * (_NE * _NRB) + e * _NRB + r, 0)),
        out_shape=jax.ShapeDtypeStruct((_NCORE * _NE * _N, _H), jnp.float32),
    )(h, W_msg, b_msg3)


def _make_scatter():
    mesh = plsc.VectorSubcoreMesh(core_axis_name="c", subcore_axis_name="s",
                                  num_cores=_NCORE, num_subcores=_NSUB)

    @functools.partial(
        pl.kernel,
        mesh=mesh,
        out_type=jax.ShapeDtypeStruct((_NCORE * _N, _H), jnp.float32),
        scratch_types=[
            pltpu.VMEM((2, 2, 1, _CH), jnp.int32),
            pltpu.VMEM((2, 2, 1, _CH), jnp.int32),
            pltpu.VMEM((_CH, _H), jnp.float32),
            pltpu.VMEM((_CH, _H), jnp.float32),
            pltpu.VMEM((_CH, _H), jnp.float32),
            pltpu.VMEM((_CH, _H), jnp.float32),
            pltpu.VMEM_SHARED((_N, _H), jnp.float32),
            pltpu.SemaphoreType.DMA,
            pltpu.SemaphoreType.DMA,
            pltpu.SemaphoreType.DMA,
            pltpu.SemaphoreType.DMA,
            pltpu.SemaphoreType.DMA,
            pltpu.SemaphoreType.DMA,
            pltpu.SemaphoreType.DMA,
            pltpu.SemaphoreType.DMA,
        ],
    )
    def scatter_kernel(ht, idxp, zrows, out,
                       ib0, ib1, r0, r1, r2, r3, acc,
                       gs0, gs1, gs2, gs3, ss0, ss1, ss2, ss3):
        c = lax.axis_index("c")
        s = lax.axis_index("s")
        pltpu.sync_copy(zrows, acc.at[pl.ds(s * _RPT, _RPT)])

        @pl.when(s == 0)
        def _():
            pltpu.sync_copy(zrows.at[pl.ds(0, _RREM)], acc.at[pl.ds(_RREM_OFF, _RREM)])

        plsc.subcore_barrier()

        tb = (c * _NSUB + s) * _NPR
        pltpu.sync_copy(idxp.at[tb], ib0)
        pltpu.async_copy(ht.at[ib0.at[0, 0, 0]], r0, gs0)
        pltpu.async_copy(ht.at[ib0.at[1, 0, 0]], r1, gs1)

        def step(j, ibA, rA, rB, gsA, gsB, ssA, ssB, ibN, rC, rD, gsC, gsD, ssC, ssD):
            pltpu.make_async_copy(ht.at[ibA.at[0, 0, 0]], rA, gsA).wait()
            pltpu.async_copy(rA, acc.at[ibA.at[0, 1, 0]], ssA, add=True)
            pltpu.make_async_copy(ht.at[ibA.at[1, 0, 0]], rB, gsB).wait()
            pltpu.async_copy(rB, acc.at[ibA.at[1, 1, 0]], ssB, add=True)

            @pl.when(j + 1 < _NPR)
            def _():
                @pl.when(j >= 1)
                def _():
                    pltpu.make_async_copy(rC, acc.at[ibN.at[0, 1, 0]], ssC).wait()
                    pltpu.make_async_copy(rD, acc.at[ibN.at[1, 1, 0]], ssD).wait()

                pltpu.sync_copy(idxp.at[tb + j + 1], ibN)
                pltpu.async_copy(ht.at[ibN.at[0, 0, 0]], rC, gsC)
                pltpu.async_copy(ht.at[ibN.at[1, 0, 0]], rD, gsD)

        def pair(j, carry):
            @pl.when(j % 2 == 0)
            def _():
                step(j, ib0, r0, r1, gs0, gs1, ss0, ss1, ib1, r2, r3, gs2, gs3, ss2, ss3)

            @pl.when(j % 2 == 1)
            def _():
                step(j, ib1, r2, r3, gs2, gs3, ss2, ss3, ib0, r0, r1, gs0, gs1, ss0, ss1)

            return carry

        lax.fori_loop(0, _NPR, pair, 0)
        pltpu.make_async_copy(r2, acc.at[ib1.at[0, 1, 0]], ss2).wait()
        pltpu.make_async_copy(r3, acc.at[ib1.at[1, 1, 0]], ss3).wait()
        pltpu.make_async_copy(r0, acc.at[ib0.at[0, 1, 0]], ss0).wait()
        pltpu.make_async_copy(r1, acc.at[ib0.at[1, 1, 0]], ss1).wait()
        plsc.subcore_barrier()
        pltpu.sync_copy(acc.at[pl.ds(s * _RPT, _RPT)],
                        out.at[pl.ds(c * _N + s * _RPT, _RPT)])

        @pl.when(s == 0)
        def _():
            pltpu.sync_copy(acc.at[pl.ds(_RREM_OFF, _RREM)],
                            out.at[pl.ds(c * _N + _RREM_OFF, _RREM)])

    return scatter_kernel


def _gru_body(alo_ref, ahi_ref, h_ref, wih_ref, whh_ref, bih_ref, bhh_ref, out_ref):
    a = jnp.concatenate([alo_ref[...], ahi_ref[...]], axis=1)
    h = h_ref[...]
    gi = lax.dot_general(a, wih_ref[...], (((1,), (1,)), ((), ())),
                         preferred_element_type=jnp.float32) + bih_ref[...]
    gh = lax.dot_general(h, whh_ref[...], (((1,), (1,)), ((), ())),
                         preferred_element_type=jnp.float32) + bhh_ref[...]
    r = jax.nn.sigmoid(gi[:, :_D] + gh[:, :_D])
    z = jax.nn.sigmoid(gi[:, _D:2 * _D] + gh[:, _D:2 * _D])
    n = jnp.tanh(gi[:, 2 * _D:] + r * gh[:, 2 * _D:])
    out_ref[...] = (1.0 - z) * n + z * h


def _gru_call(a2, h, W_ih, W_hh, bih2, bhh2):
    return pl.pallas_call(
        _gru_body,
        grid=(_NRB,),
        in_specs=[
            pl.BlockSpec((_RB, _H), lambda r: (r, 0)),
            pl.BlockSpec((_RB, _H), lambda r: (r + _NRB, 0)),
            pl.BlockSpec((_RB, _D), lambda r: (r, 0)),
            pl.BlockSpec((3 * _D, _D), lambda r: (0, 0)),
            pl.BlockSpec((3 * _D, _D), lambda r: (0, 0)),
            pl.BlockSpec((1, 3 * _D), lambda r: (0, 0)),
            pl.BlockSpec((1, 3 * _D), lambda r: (0, 0)),
        ],
        out_specs=pl.BlockSpec((_RB, _D), lambda r: (r, 0)),
        out_shape=jax.ShapeDtypeStruct((_N, _D), jnp.float32),
    )(a2, a2, h, W_ih, W_hh, bih2, bhh2)


def _conv_path(v, w3, b1, w1, b2, d):
    nt = (((1,), (0,)), ((), ()))
    y = lax.dot_general(v[0:498], w3[0], nt, preferred_element_type=jnp.float32)
    y = y + lax.dot_general(v[1:499], w3[1], nt, preferred_element_type=jnp.float32)
    y = y + lax.dot_general(v[2:500], w3[2], nt, preferred_element_type=jnp.float32)
    y = jnp.maximum(y + b1, 0.0)
    zp = jnp.maximum(y[0:497], y[1:498])
    m = jnp.max(zp[0:496].reshape(248, 2, d), axis=1)
    u = jnp.maximum(lax.dot_general(m, w1, nt, preferred_element_type=jnp.float32) + b2, 0.0)
    return jnp.max(u.reshape(124, 2, d), axis=1)


def _head_body(h_ref, x_ref, c1_ref, b1_ref, c2_ref, b2_ref,
               d1_ref, e1_ref, d2_ref, e2_ref,
               my_ref, myb_ref, mz_ref, mzb_ref, out_ref):
    h = h_ref[0]
    x = x_ref[0]
    cat = jnp.concatenate([h, x], axis=1)
    tn = (((1,), (1,)), ((), ()))
    yv = _conv_path(h, c1_ref[...], b1_ref[...], c2_ref[...], b2_ref[...], _D)
    zv = _conv_path(cat, d1_ref[...], e1_ref[...], d2_ref[...], e2_ref[...], _DC)
    ys = lax.dot_general(yv, my_ref[...], tn, preferred_element_type=jnp.float32)[:, 0:1] + myb_ref[0, 0]
    zs = lax.dot_general(zv, mz_ref[...], tn, preferred_element_type=jnp.float32)[:, 0:1] + mzb_ref[0, 0]
    avg = jnp.sum(ys * zs) * (1.0 / 124.0)
    out_ref[...] = jnp.broadcast_to(jax.nn.sigmoid(avg), (1, 1, 128))


def _head_call(h3, x3, c1T, b1, c2T, b2, d1T, e1, d2T, e2, my, myb, mz, mzb):
    full = lambda shape: pl.BlockSpec(shape, lambda b: tuple(0 for _ in shape))
    return pl.pallas_call(
        _head_body,
        grid=(_NG,),
        in_specs=[
            pl.BlockSpec((1, _NPG, _D), lambda b: (b, 0, 0)),
            pl.BlockSpec((1, _NPG, _DIN), lambda b: (b, 0, 0)),
            full((3, _D, _D)),
            full((1, _D)),
            full((_D, _D)),
            full((1, _D)),
            full((3, _DC, _DC)),
            full((1, _DC)),
            full((_DC, _DC)),
            full((1, _DC)),
            full((8, _D)),
            full((1, 1)),
            full((8, _DC)),
            full((1, 1)),
        ],
        out_specs=pl.BlockSpec((1, 1, 128), lambda b: (b, 0, 0)),
        out_shape=jax.ShapeDtypeStruct((_NG, 1, 128), jnp.float32),
    )(h3, x3, c1T, b1, c2T, b2, d1T, e1, d2T, e2, my, myb, mz, mzb)


def kernel(x, W_msg, b_msg, W_ih, W_hh, b_ih, b_hh, conv1_w, conv1_b,
           conv2_w, conv2_b, convc1_w, convc1_b, convc2_w, convc2_b,
           mlp_y_w, mlp_y_b, mlp_z_w, mlp_z_b, edge_index, edge_types):
    src = edge_index[0]
    dst = edge_index[1]
    g = edge_types * _N + src
    gt = g.reshape(_NSUB, _NPR, 2, 1, 1, _CH)
    dt = dst.reshape(_NSUB, _NPR, 2, 1, 1, _CH)
    idxp = jnp.concatenate([
        jnp.concatenate([gt, dt], axis=3),
        jnp.concatenate([gt + _NE * _N, dt], axis=3),
    ], axis=0).reshape(_NCORE * _NSUB * _NPR, 2, 2, 1, _CH)
    zrows = jnp.zeros((_RPT, _H), jnp.float32)
    b_msg3 = b_msg.reshape(_NE * _NCORE, 1, _H)
    bih2 = b_ih.reshape(1, 3 * _D)
    bhh2 = b_hh.reshape(1, 3 * _D)
    scatter = _make_scatter()

    h = jnp.pad(x, ((0, 0), (0, _D - _DIN)))
    for _ in range(_STEPS):
        ht = _msg_call(h, W_msg, b_msg3)
        a2 = scatter(ht, idxp, zrows)
        h = _gru_call(a2, h, W_ih, W_hh, bih2, bhh2)

    c1T = jnp.transpose(conv1_w, (2, 1, 0))
    c2T = jnp.transpose(conv2_w[:, :, 0], (1, 0))
    d1T = jnp.transpose(convc1_w, (2, 1, 0))
    d2T = jnp.transpose(convc2_w[:, :, 0], (1, 0))
    myp = jnp.pad(mlp_y_w, ((0, 7), (0, 0)))
    mzp = jnp.pad(mlp_z_w, ((0, 7), (0, 0)))
    out3 = _head_call(h.reshape(_NG, _NPG, _D), x.reshape(_NG, _NPG, _DIN),
                      c1T, conv1_b.reshape(1, _D), c2T, conv2_b.reshape(1, _D),
                      d1T, convc1_b.reshape(1, _DC), d2T, convc2_b.reshape(1, _DC),
                      myp, mlp_y_b.reshape(1, 1), mzp, mlp_z_b.reshape(1, 1))
    return out3[:, 0, 0]


# speedup vs baseline: 1.3286x; 1.3286x over previous
import functools

import jax
import jax.numpy as jnp
from jax import lax
from jax.experimental import pallas as pl
from jax.experimental.pallas import tpu as pltpu
from jax.experimental.pallas import tpu_sc as plsc

_N = 10000
_E = 320000
_DIN = 128
_D = 256
_NE = 4
_STEPS = 8
_NG = 20
_NPG = 500
_DC = _DIN + _D

_H = 128
_NCORE = 2
_NSUB = 16
_EPT = _E // _NSUB
_CH = 128
_NFULL = _EPT // _CH
_TAIL = _EPT - _NFULL * _CH
_NPAIR = _NFULL // 2
_RPT = 624
_RREM = _N - _NSUB * _RPT
_RREM_OFF = _NSUB * _RPT

_RB = 2000
_NRB = _N // _RB


def _msg_body(h_ref, w_ref, b_ref, out_ref):
    w = w_ref[0]
    acc = lax.dot_general(h_ref[...], w, (((1,), (1,)), ((), ())),
                          preferred_element_type=jnp.float32)
    out_ref[...] = acc + b_ref[0]


def _msg_call(h, W_msg, b_msg3):
    return pl.pallas_call(
        _msg_body,
        grid=(_NRB, _NCORE, _NE),
        in_specs=[
            pl.BlockSpec((_RB, _D), lambda r, c, e: (r, 0)),
            pl.BlockSpec((1, _H, _D), lambda r, c, e: (e, c, 0)),
            pl.BlockSpec((1, 1, _H), lambda r, c, e: (e * _NCORE + c, 0, 0)),
        ],
        out_specs=pl.BlockSpec((_RB, _H), lambda r, c, e: (c * (_NE * _NRB) + e * _NRB + r, 0)),
        out_shape=jax.ShapeDtypeStruct((_NCORE * _NE * _N, _H), jnp.float32),
    )(h, W_msg, b_msg3)


def _make_scatter():
    mesh = plsc.VectorSubcoreMesh(core_axis_name="c", subcore_axis_name="s",
                                  num_cores=_NCORE, num_subcores=_NSUB)

    @functools.partial(
        pl.kernel,
        mesh=mesh,
        out_type=jax.ShapeDtypeStruct((_NCORE * _N, _H), jnp.float32),
        scratch_types=[
            pltpu.VMEM((1, _CH), jnp.int32),
            pltpu.VMEM((1, _CH), jnp.int32),
            pltpu.VMEM((1, _CH), jnp.int32),
            pltpu.VMEM((1, _CH), jnp.int32),
            pltpu.VMEM((_CH, _H), jnp.float32),
            pltpu.VMEM((_CH, _H), jnp.float32),
            pltpu.VMEM((_TAIL,), jnp.int32),
            pltpu.VMEM((_TAIL,), jnp.int32),
            pltpu.VMEM((_TAIL, _H), jnp.float32),
            pltpu.VMEM_SHARED((_N, _H), jnp.float32),
            pltpu.SemaphoreType.DMA,
            pltpu.SemaphoreType.DMA,
            pltpu.SemaphoreType.DMA,
            pltpu.SemaphoreType.DMA,
            pltpu.SemaphoreType.DMA,
            pltpu.SemaphoreType.DMA,
        ],
    )
    def scatter_kernel(ht, gidxp, didxp, g2, dst, zrows, out,
                       gib0, gib1, dib0, dib1, rows0, rows1,
                       gbuf_t, dbuf_t, rows_t,
                       acc, sem0, sem1, gi0, gi1, di0, di1):
        c = lax.axis_index("c")
        s = lax.axis_index("s")
        pltpu.sync_copy(zrows, acc.at[pl.ds(s * _RPT, _RPT)])

        @pl.when(s == 0)
        def _():
            pltpu.sync_copy(zrows.at[pl.ds(0, _RREM)], acc.at[pl.ds(_RREM_OFF, _RREM)])

        plsc.subcore_barrier()

        tg = (c * _NSUB + s) * _NFULL
        td = s * _NFULL
        pltpu.async_copy(gidxp.at[tg], gib0, gi0)
        pltpu.make_async_copy(gidxp.at[tg], gib0, gi0).wait()
        pltpu.async_copy(ht.at[gib0.at[0]], rows0, sem0)
        pltpu.async_copy(gidxp.at[tg + 1], gib1, gi1)
        pltpu.async_copy(didxp.at[td], dib0, di0)
        pltpu.async_copy(didxp.at[td + 1], dib1, di1)

        def pair(j, carry):
            c0 = 2 * j
            c1 = c0 + 1
            pltpu.make_async_copy(gidxp.at[tg + c1], gib1, gi1).wait()
            pltpu.async_copy(ht.at[gib1.at[0]], rows1, sem1)

            pltpu.make_async_copy(ht.at[gib0.at[0]], rows0, sem0).wait()

            @pl.when(j < _NPAIR - 1)
            def _():
                pltpu.async_copy(gidxp.at[tg + c0 + 2], gib0, gi0)

            pltpu.make_async_copy(didxp.at[td + c0], dib0, di0).wait()
            pltpu.sync_copy(rows0, acc.at[dib0.at[0]], add=True)

            @pl.when(j < _NPAIR - 1)
            def _():
                pltpu.async_copy(didxp.at[td + c0 + 2], dib0, di0)
                pltpu.make_async_copy(gidxp.at[tg + c0 + 2], gib0, gi0).wait()
                pltpu.async_copy(ht.at[gib0.at[0]], rows0, sem0)

            pltpu.make_async_copy(ht.at[gib1.at[0]], rows1, sem1).wait()

            @pl.when(j < _NPAIR - 1)
            def _():
                pltpu.async_copy(gidxp.at[tg + c1 + 2], gib1, gi1)

            pltpu.make_async_copy(didxp.at[td + c1], dib1, di1).wait()
            pltpu.sync_copy(rows1, acc.at[dib1.at[0]], add=True)

            @pl.when(j < _NPAIR - 1)
            def _():
                pltpu.async_copy(didxp.at[td + c1 + 2], dib1, di1)

            return carry

        lax.fori_loop(0, _NPAIR, pair, 0)
        st = s * _EPT + _NFULL * _CH
        pltpu.sync_copy(g2.at[pl.ds(c * _E + st, _TAIL)], gbuf_t)
        pltpu.sync_copy(dst.at[pl.ds(st, _TAIL)], dbuf_t)
        pltpu.async_copy(ht.at[gbuf_t], rows_t, sem0).wait()
        pltpu.sync_copy(rows_t, acc.at[dbuf_t], add=True)
        plsc.subcore_barrier()
        pltpu.sync_copy(acc.at[pl.ds(s * _RPT, _RPT)],
                        out.at[pl.ds(c * _N + s * _RPT, _RPT)])

        @pl.when(s == 0)
        def _():
            pltpu.sync_copy(acc.at[pl.ds(_RREM_OFF, _RREM)],
                            out.at[pl.ds(c * _N + _RREM_OFF, _RREM)])

    return scatter_kernel


def _gru_body(alo_ref, ahi_ref, h_ref, wih_ref, whh_ref, bih_ref, bhh_ref, out_ref):
    a = jnp.concatenate([alo_ref[...], ahi_ref[...]], axis=1)
    h = h_ref[...]
    gi = lax.dot_general(a, wih_ref[...], (((1,), (1,)), ((), ())),
                         preferred_element_type=jnp.float32) + bih_ref[...]
    gh = lax.dot_general(h, whh_ref[...], (((1,), (1,)), ((), ())),
                         preferred_element_type=jnp.float32) + bhh_ref[...]
    r = jax.nn.sigmoid(gi[:, :_D] + gh[:, :_D])
    z = jax.nn.sigmoid(gi[:, _D:2 * _D] + gh[:, _D:2 * _D])
    n = jnp.tanh(gi[:, 2 * _D:] + r * gh[:, 2 * _D:])
    out_ref[...] = (1.0 - z) * n + z * h


def _gru_call(a2, h, W_ih, W_hh, bih2, bhh2):
    return pl.pallas_call(
        _gru_body,
        grid=(_NRB,),
        in_specs=[
            pl.BlockSpec((_RB, _H), lambda r: (r, 0)),
            pl.BlockSpec((_RB, _H), lambda r: (r + _NRB, 0)),
            pl.BlockSpec((_RB, _D), lambda r: (r, 0)),
            pl.BlockSpec((3 * _D, _D), lambda r: (0, 0)),
            pl.BlockSpec((3 * _D, _D), lambda r: (0, 0)),
            pl.BlockSpec((1, 3 * _D), lambda r: (0, 0)),
            pl.BlockSpec((1, 3 * _D), lambda r: (0, 0)),
        ],
        out_specs=pl.BlockSpec((_RB, _D), lambda r: (r, 0)),
        out_shape=jax.ShapeDtypeStruct((_N, _D), jnp.float32),
    )(a2, a2, h, W_ih, W_hh, bih2, bhh2)


def _conv_path(v, w3, b1, w1, b2, d):
    nt = (((1,), (0,)), ((), ()))
    y = lax.dot_general(v[0:498], w3[0], nt, preferred_element_type=jnp.float32)
    y = y + lax.dot_general(v[1:499], w3[1], nt, preferred_element_type=jnp.float32)
    y = y + lax.dot_general(v[2:500], w3[2], nt, preferred_element_type=jnp.float32)
    y = jnp.maximum(y + b1, 0.0)
    zp = jnp.maximum(y[0:497], y[1:498])
    m = jnp.max(zp[0:496].reshape(248, 2, d), axis=1)
    u = jnp.maximum(lax.dot_general(m, w1, nt, preferred_element_type=jnp.float32) + b2, 0.0)
    return jnp.max(u.reshape(124, 2, d), axis=1)


def _head_body(h_ref, x_ref, c1_ref, b1_ref, c2_ref, b2_ref,
               d1_ref, e1_ref, d2_ref, e2_ref,
               my_ref, myb_ref, mz_ref, mzb_ref, out_ref):
    h = h_ref[0]
    x = x_ref[0]
    cat = jnp.concatenate([h, x], axis=1)
    tn = (((1,), (1,)), ((), ()))
    yv = _conv_path(h, c1_ref[...], b1_ref[...], c2_ref[...], b2_ref[...], _D)
    zv = _conv_path(cat, d1_ref[...], e1_ref[...], d2_ref[...], e2_ref[...], _DC)
    ys = lax.dot_general(yv, my_ref[...], tn, preferred_element_type=jnp.float32)[:, 0:1] + myb_ref[0, 0]
    zs = lax.dot_general(zv, mz_ref[...], tn, preferred_element_type=jnp.float32)[:, 0:1] + mzb_ref[0, 0]
    avg = jnp.sum(ys * zs) * (1.0 / 124.0)
    out_ref[...] = jnp.broadcast_to(jax.nn.sigmoid(avg), (1, 1, 128))


def _head_call(h3, x3, c1T, b1, c2T, b2, d1T, e1, d2T, e2, my, myb, mz, mzb):
    full = lambda shape: pl.BlockSpec(shape, lambda b: tuple(0 for _ in shape))
    return pl.pallas_call(
        _head_body,
        grid=(_NG,),
        in_specs=[
            pl.BlockSpec((1, _NPG, _D), lambda b: (b, 0, 0)),
            pl.BlockSpec((1, _NPG, _DIN), lambda b: (b, 0, 0)),
            full((3, _D, _D)),
            full((1, _D)),
            full((_D, _D)),
            full((1, _D)),
            full((3, _DC, _DC)),
            full((1, _DC)),
            full((_DC, _DC)),
            full((1, _DC)),
            full((8, _D)),
            full((1, 1)),
            full((8, _DC)),
            full((1, 1)),
        ],
        out_specs=pl.BlockSpec((1, 1, 128), lambda b: (b, 0, 0)),
        out_shape=jax.ShapeDtypeStruct((_NG, 1, 128), jnp.float32),
    )(h3, x3, c1T, b1, c2T, b2, d1T, e1, d2T, e2, my, myb, mz, mzb)


def kernel(x, W_msg, b_msg, W_ih, W_hh, b_ih, b_hh, conv1_w, conv1_b,
           conv2_w, conv2_b, convc1_w, convc1_b, convc2_w, convc2_b,
           mlp_y_w, mlp_y_b, mlp_z_w, mlp_z_b, edge_index, edge_types):
    src = edge_index[0]
    dst = edge_index[1]
    g = edge_types * _N + src
    g2 = jnp.concatenate([g, g + _NE * _N])
    gc0 = g.reshape(_NSUB, _EPT)[:, :_NFULL * _CH].reshape(_NSUB * _NFULL, 1, _CH)
    gidxp = jnp.concatenate([gc0, gc0 + _NE * _N], axis=0)
    didxp = dst.reshape(_NSUB, _EPT)[:, :_NFULL * _CH].reshape(_NSUB * _NFULL, 1, _CH)
    zrows = jnp.zeros((_RPT, _H), jnp.float32)
    b_msg3 = b_msg.reshape(_NE * _NCORE, 1, _H)
    bih2 = b_ih.reshape(1, 3 * _D)
    bhh2 = b_hh.reshape(1, 3 * _D)
    scatter = _make_scatter()

    h = jnp.pad(x, ((0, 0), (0, _D - _DIN)))
    for _ in range(_STEPS):
        ht = _msg_call(h, W_msg, b_msg3)
        a2 = scatter(ht, gidxp, didxp, g2, dst, zrows)
        h = _gru_call(a2, h, W_ih, W_hh, bih2, bhh2)

    c1T = jnp.transpose(conv1_w, (2, 1, 0))
    c2T = jnp.transpose(conv2_w[:, :, 0], (1, 0))
    d1T = jnp.transpose(convc1_w, (2, 1, 0))
    d2T = jnp.transpose(convc2_w[:, :, 0], (1, 0))
    myp = jnp.pad(mlp_y_w, ((0, 7), (0, 0)))
    mzp = jnp.pad(mlp_z_w, ((0, 7), (0, 0)))
    out3 = _head_call(h.reshape(_NG, _NPG, _D), x.reshape(_NG, _NPG, _DIN),
                      c1T, conv1_b.reshape(1, _D), c2T, conv2_b.reshape(1, _D),
                      d1T, convc1_b.reshape(1, _DC), d2T, convc2_b.reshape(1, _DC),
                      myp, mlp_y_b.reshape(1, 1), mzp, mlp_z_b.reshape(1, 1))
    return out3[:, 0, 0]
